# Initial kernel scaffold; baseline (speedup 1.0000x reference)
import jax
import jax.numpy as jnp
from jax.experimental import pallas as pl


def kernel(x):
    raise NotImplementedError("write your pallas kernel here")


# speedup vs baseline: 10.2240x; 10.2240x over previous
import functools

import jax
import jax.numpy as jnp
from jax.experimental import pallas as pl

NB = 11
P = 4
H = 512
W = 512
HO = 127
WO = 127
HC = HO * P
WC = WO * P

_ATAN_COEFFS = (
    1.0, -0.33333278, 0.19998075, -0.14260016, 0.10932341,
    -0.08349725, 0.057089556, -0.030351864, 0.01048765, -0.0017011701,
)


def _atan2(y, x):
    ay = jnp.abs(y)
    ax = jnp.abs(x)
    mn = jnp.minimum(ay, ax)
    mx = jnp.maximum(ay, ax)
    t = mn / jnp.where(mx > 0, mx, jnp.float32(1.0))
    u = t * t
    acc = jnp.full_like(u, _ATAN_COEFFS[-1])
    for c in _ATAN_COEFFS[-2::-1]:
        acc = acc * u + jnp.float32(c)
    r = acc * t
    r = jnp.where(ay > ax, jnp.float32(jnp.pi / 2) - r, r)
    r = jnp.where(x < 0, jnp.float32(jnp.pi) - r, r)
    return jnp.where(y < 0, -r, r)


def _hog_kernel(x_ref, o_ref):
    xb = x_ref[0, 0].astype(jnp.bfloat16).astype(jnp.float32)

    gx = xb[1:HC + 1, 0:WC] - xb[1:HC + 1, 2:WC + 2]
    gy = xb[0:HC, 1:WC + 1] - xb[2:HC + 2, 1:WC + 1]

    mag = jnp.sqrt(gx * gx + gy * gy)
    phase = _atan2(gx, gy)
    t = phase / jnp.float32(jnp.pi) * jnp.float32(NB)

    f = jnp.floor(t)
    c = jnp.ceil(t)
    f = jnp.where(f < 0, f + NB, f)
    f = jnp.where(f >= NB, f - NB, f)
    c = jnp.where(c < 0, c + NB, c)
    c = jnp.where(c >= NB, c - NB, c)

    one_minus_mag = 1.0 - mag

    row = jax.lax.broadcasted_iota(jnp.int32, (HC, WO), 0)
    col = jax.lax.broadcasted_iota(jnp.int32, (HC, WO), 1)
    pw = jnp.where(row // P == col, jnp.float32(0.25), jnp.float32(0.0))
    rowt = jax.lax.broadcasted_iota(jnp.int32, (HO, HC), 0)
    colt = jax.lax.broadcasted_iota(jnp.int32, (HO, HC), 1)
    pht = jnp.where(colt // P == rowt, jnp.float32(0.25), jnp.float32(0.0))

    for k in range(NB):
        fk = jnp.float32(k)
        mk = jnp.where(f == fk, mag, 0.0) + jnp.where(c == fk, one_minus_mag, 0.0)
        a = jax.lax.dot(mk, pw, precision=jax.lax.Precision.HIGHEST)
        o_ref[0, k] = jax.lax.dot(pht, a, precision=jax.lax.Precision.HIGHEST)


def kernel(x):
    n = x.shape[0]
    return pl.pallas_call(
        _hog_kernel,
        grid=(n,),
        in_specs=[pl.BlockSpec((1, 1, H, W), lambda b: (b, 0, 0, 0))],
        out_specs=pl.BlockSpec((1, NB, HO, WO), lambda b: (b, 0, 0, 0)),
        out_shape=jax.ShapeDtypeStruct((n, NB, HO, WO), jnp.float32),
    )(x)


# speedup vs baseline: 29.9932x; 2.9336x over previous
import functools

import jax
import jax.numpy as jnp
from jax.experimental import pallas as pl

NB = 11
P = 4
H = 512
W = 512
HO = 127
WO = 127
HC = HO * P
WC = WO * P

_ATAN_COEFFS = (
    1.0, -0.33333278, 0.19998075, -0.14260016, 0.10932341,
    -0.08349725, 0.057089556, -0.030351864, 0.01048765, -0.0017011701,
)


def _atan2(y, x):
    ay = jnp.abs(y)
    ax = jnp.abs(x)
    mn = jnp.minimum(ay, ax)
    mx = jnp.maximum(ay, ax)
    t = mn / jnp.where(mx > 0, mx, jnp.float32(1.0))
    u = t * t
    acc = jnp.full_like(u, _ATAN_COEFFS[-1])
    for c in _ATAN_COEFFS[-2::-1]:
        acc = acc * u + jnp.float32(c)
    r = acc * t
    r = jnp.where(ay > ax, jnp.float32(jnp.pi / 2) - r, r)
    r = jnp.where(x < 0, jnp.float32(jnp.pi) - r, r)
    return jnp.where(y < 0, -r, r)


def _hog_kernel(x_ref, o_ref):
    xb = x_ref[0, 0].astype(jnp.bfloat16).astype(jnp.float32)

    gx = xb[1:HC + 1, 0:WC] - xb[1:HC + 1, 2:WC + 2]
    gy = xb[0:HC, 1:WC + 1] - xb[2:HC + 2, 1:WC + 1]

    mag = jnp.sqrt(gx * gx + gy * gy)
    phase = jnp.arctan2(gx, gy)
    t = phase / jnp.float32(jnp.pi) * jnp.float32(NB)

    f = jnp.floor(t)
    c = jnp.ceil(t)
    f = jnp.where(f < 0, f + NB, f)
    f = jnp.where(f >= NB, f - NB, f)
    c = jnp.where(c < 0, c + NB, c)
    c = jnp.where(c >= NB, c - NB, c)

    one_minus_mag = 1.0 - mag

    row = jax.lax.broadcasted_iota(jnp.int32, (HC, WO), 0)
    col = jax.lax.broadcasted_iota(jnp.int32, (HC, WO), 1)
    pw = jnp.where(row // P == col, jnp.float32(0.25), jnp.float32(0.0))
    rowt = jax.lax.broadcasted_iota(jnp.int32, (HO, HC), 0)
    colt = jax.lax.broadcasted_iota(jnp.int32, (HO, HC), 1)
    pht = jnp.where(colt // P == rowt, jnp.float32(0.25), jnp.float32(0.0))

    for k in range(NB):
        fk = jnp.float32(k)
        mk = jnp.where(f == fk, mag, 0.0) + jnp.where(c == fk, one_minus_mag, 0.0)
        a = jax.lax.dot(mk, pw)
        o_ref[0, k] = jax.lax.dot(pht, a)


def kernel(x):
    n = x.shape[0]
    return pl.pallas_call(
        _hog_kernel,
        grid=(n,),
        in_specs=[pl.BlockSpec((1, 1, H, W), lambda b: (b, 0, 0, 0))],
        out_specs=pl.BlockSpec((1, NB, HO, WO), lambda b: (b, 0, 0, 0)),
        out_shape=jax.ShapeDtypeStruct((n, NB, HO, WO), jnp.float32),
    )(x)
